# baseline (speedup 1.0000x reference)
import jax
import jax.numpy as jnp
from jax import lax
from jax.experimental import pallas as pl
from jax.experimental.pallas import tpu as pltpu

B, C, L, K = 32, 256, 1024, 1024
NB = 2
ROWS = B * C
TR = 512


def _gelu(t):
    return 0.5 * t * (1.0 + lax.erf(t * jnp.float32(0.70710678118654752440)))


def _vq_body(z_ref, E_ref, enc_ref, q_ref, loss_ref, ppl_ref,
             cnt_scr, loss_scr):
    i = pl.program_id(0)
    zb = z_ref[0]
    E = E_ref[...]
    En = jnp.sum(E * E, axis=1, keepdims=True)

    @pl.when(i == 0)
    def _init():
        cnt_scr[...] = jnp.zeros_like(cnt_scr)
        loss_scr[0, 0] = 0.0

    flat2 = jnp.sum(zb * zb, axis=0, keepdims=True)
    Mt = jnp.dot(E, zb)
    dist = (flat2 + En) - 2.0 * Mt
    kidx = lax.broadcasted_iota(jnp.int32, (K, L), 0)
    dmin = jnp.min(dist, axis=0, keepdims=True)
    sel = jnp.where(dist == dmin, kidx, K)
    kmin = jnp.min(sel, axis=0, keepdims=True)
    enc_t = (kidx == kmin).astype(jnp.float32)
    qb = lax.dot_general(E, enc_t, (((0,), (0,)), ((), ())))
    q_ref[0] = qb
    kmin_col = kmin.reshape(L, 1)
    lidx = lax.broadcasted_iota(jnp.int32, (L, K), 1)
    enc_ref[...] = (lidx == kmin_col).astype(jnp.float32)
    diff = qb - zb
    loss_scr[0, 0] += jnp.sum(diff * diff)
    cnt_scr[...] += jnp.sum(enc_t, axis=1, keepdims=True)

    @pl.when(i == pl.num_programs(0) - 1)
    def _fin():
        mse = loss_scr[0, 0] / jnp.float32(B * C * L)
        loss_ref[...] = jnp.full((1, 1), mse + 0.25 * mse, jnp.float32)
        avg = cnt_scr[...] / jnp.float32(B * L)
        s = jnp.sum(avg * jnp.log(avg + 1e-10), axis=0, keepdims=True)
        ppl_ref[...] = jnp.exp(-s)


def _dec_body(q_ref, Wd0_ref, bd0_ref, Wr1d_ref, br1d_ref, Wr2d_ref, br2d_ref,
              Wd1_ref, bd1_ref, Wd2_ref, bd2_ref, xr_ref):
    d = _gelu(jnp.dot(q_ref[...], Wd0_ref[...]) + bd0_ref[...])
    r = jnp.maximum(jnp.dot(d, Wr1d_ref[...]) + br1d_ref[...], 0.0)
    r = jnp.maximum(jnp.dot(r, Wr2d_ref[...]) + br2d_ref[...], 0.0)
    d = d + r
    d = _gelu(jnp.dot(d, Wd1_ref[...]) + bd1_ref[...])
    xr_ref[...] = _gelu(jnp.dot(d, Wd2_ref[...]) + bd2_ref[...])


def kernel(x, We1, be1, We2, be2, Wr1e, br1e, Wr2e, br2e, Wpre, bpre, E,
           Wd0, bd0, Wr1d, br1d, Wr2d, br2d, Wd1, bd1, Wd2, bd2):
    f32 = jnp.float32
    gelu = lambda t: jax.nn.gelu(t, approximate=False)
    h = gelu(x @ We1 + be1)
    h = gelu(h @ We2 + be2)
    r = jax.nn.relu(h @ Wr1e + br1e)
    r = jax.nn.relu(r @ Wr2e + br2e)
    z = (h + r) @ Wpre + bpre
    z = lax.optimization_barrier(z)

    def full(a):
        return pl.BlockSpec(a.shape, lambda i: (0,) * a.ndim)

    enc, quant, loss, ppl = pl.pallas_call(
        _vq_body,
        grid=(B,),
        in_specs=[pl.BlockSpec((1, C, L), lambda i: (i, 0, 0)), full(E)],
        out_specs=[
            pl.BlockSpec((L, K), lambda i: (i, 0)),
            pl.BlockSpec((1, C, L), lambda i: (i, 0, 0)),
            pl.BlockSpec((1, 1), lambda i: (0, 0)),
            pl.BlockSpec((1, 1), lambda i: (0, 0)),
        ],
        out_shape=[
            jax.ShapeDtypeStruct((B * L, K), f32),
            jax.ShapeDtypeStruct((B, C, L), f32),
            jax.ShapeDtypeStruct((1, 1), f32),
            jax.ShapeDtypeStruct((1, 1), f32),
        ],
        scratch_shapes=[pltpu.VMEM((K, 1), f32), pltpu.SMEM((1, 1), f32)],
        compiler_params=pltpu.CompilerParams(
            dimension_semantics=("arbitrary",)),
    )(z, E)

    dec_ws = [Wd0, bd0.reshape(1, -1), Wr1d, br1d.reshape(1, -1),
              Wr2d, br2d.reshape(1, -1), Wd1, bd1.reshape(1, -1),
              Wd2, bd2.reshape(1, -1)]
    xr = pl.pallas_call(
        _dec_body,
        grid=(ROWS // TR,),
        in_specs=[pl.BlockSpec((TR, L), lambda i: (i, 0))]
                 + [full(w) for w in dec_ws],
        out_specs=pl.BlockSpec((TR, 1024), lambda i: (i, 0)),
        out_shape=jax.ShapeDtypeStruct((ROWS, 1024), f32),
        compiler_params=pltpu.CompilerParams(
            dimension_semantics=("parallel",)),
    )(quant.reshape(ROWS, L), *dec_ws)

    return (loss.reshape(()), xr.reshape(B, C, 1024), ppl.reshape(()),
            enc, quant)


# speedup vs baseline: 1.0014x; 1.0014x over previous
import jax
import jax.numpy as jnp
from jax import lax
from jax.experimental import pallas as pl
from jax.experimental.pallas import tpu as pltpu

B, C, L, K = 32, 256, 1024, 1024
ROWS = B * C
TR = 512


def _gelu(t):
    return 0.5 * t * (1.0 + lax.erf(t * jnp.float32(0.70710678118654752440)))


def _vq_body(z_ref, E_ref, enc_ref, q_ref, loss_ref, ppl_ref,
             cnt_scr, loss_scr):
    i = pl.program_id(0)
    zb = z_ref[0]
    E = E_ref[...]
    En = jnp.sum(E * E, axis=1, keepdims=True)

    @pl.when(i == 0)
    def _init():
        cnt_scr[...] = jnp.zeros_like(cnt_scr)
        loss_scr[0, 0] = 0.0

    flat2 = jnp.sum(zb * zb, axis=0, keepdims=True)
    Mt = jnp.dot(E, zb)
    dist = (flat2 + En) - 2.0 * Mt
    kidx = lax.broadcasted_iota(jnp.int32, (K, L), 0)
    dmin = jnp.min(dist, axis=0, keepdims=True)
    sel = jnp.where(dist == dmin, kidx, K)
    kmin = jnp.min(sel, axis=0, keepdims=True)
    enc_t = (kidx == kmin).astype(jnp.float32)
    qb = lax.dot_general(E, enc_t, (((0,), (0,)), ((), ())))
    q_ref[0] = qb
    kmin_col = kmin.reshape(L, 1)
    lidx = lax.broadcasted_iota(jnp.int32, (L, K), 1)
    enc_ref[...] = (lidx == kmin_col).astype(jnp.float32)
    diff = qb - zb
    loss_scr[0, 0] += jnp.sum(diff * diff)
    cnt_scr[...] += jnp.sum(enc_t, axis=1, keepdims=True)

    @pl.when(i == pl.num_programs(0) - 1)
    def _fin():
        mse = loss_scr[0, 0] / jnp.float32(B * C * L)
        loss_ref[...] = jnp.full((1, 1), mse + 0.25 * mse, jnp.float32)
        avg = cnt_scr[...] / jnp.float32(B * L)
        s = jnp.sum(avg * jnp.log(avg + 1e-10), axis=0, keepdims=True)
        ppl_ref[...] = jnp.exp(-s)


def _dec_body(q_ref, Wd0_ref, bd0_ref, Wr1d_ref, br1d_ref, Wr2d_ref, br2d_ref,
              Wd1_ref, bd1_ref, Wd2_ref, bd2_ref, xr_ref):
    d = _gelu(jnp.dot(q_ref[...], Wd0_ref[...]) + bd0_ref[...])
    r = jnp.maximum(jnp.dot(d, Wr1d_ref[...]) + br1d_ref[...], 0.0)
    r = jnp.maximum(jnp.dot(r, Wr2d_ref[...]) + br2d_ref[...], 0.0)
    d = d + r
    d = _gelu(jnp.dot(d, Wd1_ref[...]) + bd1_ref[...])
    xr_ref[...] = _gelu(jnp.dot(d, Wd2_ref[...]) + bd2_ref[...])


def kernel(x, We1, be1, We2, be2, Wr1e, br1e, Wr2e, br2e, Wpre, bpre, E,
           Wd0, bd0, Wr1d, br1d, Wr2d, br2d, Wd1, bd1, Wd2, bd2):
    f32 = jnp.float32
    gelu = lambda t: jax.nn.gelu(t, approximate=False)
    h = gelu(x @ We1 + be1)
    h = gelu(h @ We2 + be2)
    r = jax.nn.relu(h @ Wr1e + br1e)
    r = jax.nn.relu(r @ Wr2e + br2e)
    z = (h + r) @ Wpre + bpre
    z = lax.optimization_barrier(z)

    def full(a):
        return pl.BlockSpec(a.shape, lambda i: (0,) * a.ndim)

    enc, quant, loss, ppl = pl.pallas_call(
        _vq_body,
        grid=(B,),
        in_specs=[pl.BlockSpec((1, C, L), lambda i: (i, 0, 0)), full(E)],
        out_specs=[
            pl.BlockSpec((L, K), lambda i: (i, 0)),
            pl.BlockSpec((1, C, L), lambda i: (i, 0, 0)),
            pl.BlockSpec((1, 1), lambda i: (0, 0)),
            pl.BlockSpec((1, 1), lambda i: (0, 0)),
        ],
        out_shape=[
            jax.ShapeDtypeStruct((B * L, K), f32),
            jax.ShapeDtypeStruct((B, C, L), f32),
            jax.ShapeDtypeStruct((1, 1), f32),
            jax.ShapeDtypeStruct((1, 1), f32),
        ],
        scratch_shapes=[pltpu.VMEM((K, 1), f32), pltpu.SMEM((1, 1), f32)],
        compiler_params=pltpu.CompilerParams(
            dimension_semantics=("arbitrary",)),
    )(z, E)

    dec_ws = [Wd0, bd0.reshape(1, -1), Wr1d, br1d.reshape(1, -1),
              Wr2d, br2d.reshape(1, -1), Wd1, bd1.reshape(1, -1),
              Wd2, bd2.reshape(1, -1)]
    xr = pl.pallas_call(
        _dec_body,
        grid=(ROWS // TR,),
        in_specs=[pl.BlockSpec((TR, L), lambda i: (i, 0))]
                 + [full(w) for w in dec_ws],
        out_specs=pl.BlockSpec((TR, 1024), lambda i: (i, 0)),
        out_shape=jax.ShapeDtypeStruct((ROWS, 1024), f32),
        compiler_params=pltpu.CompilerParams(
            dimension_semantics=("parallel",)),
    )(quant.reshape(ROWS, L), *dec_ws)

    return (loss.reshape(()), xr.reshape(B, C, 1024), ppl.reshape(()),
            enc, quant)
